# speedup vs baseline: 2.5128x; 2.5128x over previous
import functools

import jax
import jax.numpy as jnp
from jax import lax
from jax.experimental import pallas as pl
from jax.experimental.pallas import tpu as pltpu
from jax.experimental.pallas import tpu_sc as plsc

N_NODES = 100000
D_IN = 128
D_OUT = 384
B = 16000
NUM_WALKS = 16

NC, NS, L = 2, 16, 16
NW = NC * NS
BP = 16384
RPW = BP // NW
SUB = 8
SUPER = 4
NSUB = RPW // SUB
ROWS_SUPER = SUB * SUPER
NSUPER = RPW // ROWS_SUPER


def _sc_aggregate(feat, nodes3, adj3, dis3):
    mesh = plsc.VectorSubcoreMesh(core_axis_name="c", subcore_axis_name="s")

    @functools.partial(
        pl.kernel,
        mesh=mesh,
        out_type=[jax.ShapeDtypeStruct((BP, D_IN), jnp.float32)] * 3,
        scratch_types=[
            pltpu.VMEM((NSUPER, ROWS_SUPER), jnp.int32),
            pltpu.VMEM((NSUB, SUB * 16), jnp.int32),
            pltpu.VMEM((NSUB, SUB * 16), jnp.int32),
            pltpu.VMEM((SUB * 16, D_IN), jnp.float32),
            pltpu.VMEM((SUB * 16, D_IN), jnp.float32),
            pltpu.VMEM((ROWS_SUPER, D_IN), jnp.float32),
            pltpu.VMEM((ROWS_SUPER, D_IN), jnp.float32),
            pltpu.VMEM((ROWS_SUPER, D_IN), jnp.float32),
            pltpu.SemaphoreType.DMA,
            pltpu.SemaphoreType.DMA,
            pltpu.SemaphoreType.DMA,
            pltpu.SemaphoreType.DMA,
        ],
    )
    def sc_kernel(feat_hbm, nodes_hbm, adj_hbm, dis_hbm,
                  outS, outA, outD,
                  idxS, idxA, idxD, bufA, bufD, stS, stA, stD,
                  semS, semA, semD, semO):
        wid = lax.axis_index("s") * NC + lax.axis_index("c")

        pltpu.sync_copy(nodes_hbm.at[wid], idxS)
        pltpu.sync_copy(adj_hbm.at[wid], idxA)
        pltpu.sync_copy(dis_hbm.at[wid], idxD)

        def sub_body(k, _):
            s = k // SUPER
            k4 = k % SUPER

            @pl.when(k4 == 0)
            def _start_self():
                pltpu.async_copy(feat_hbm.at[idxS.at[s]], stS, semS)

            cpA = pltpu.async_copy(feat_hbm.at[idxA.at[k]], bufA, semA)
            cpD = pltpu.async_copy(feat_hbm.at[idxD.at[k]], bufD, semD)
            cpA.wait()
            cpD.wait()

            def row_body(r, _):
                srow = k4 * SUB + r
                base = r * 16
                for cc in range(D_IN // L):
                    col = pl.ds(cc * L, L)
                    accA = bufA[base, col]
                    accD = bufD[base, col]
                    for j in range(1, 16):
                        accA = accA + bufA[base + j, col]
                        accD = accD + bufD[base + j, col]
                    stA[srow, col] = accA
                    stD[srow, col] = accD
                return 0

            lax.fori_loop(0, SUB, row_body, 0)

            @pl.when(k4 == SUPER - 1)
            def _flush():
                row0 = wid * RPW + s * ROWS_SUPER
                pltpu.make_async_copy(feat_hbm.at[idxS.at[s]], stS, semS).wait()
                cS = pltpu.async_copy(stS, outS.at[pl.ds(row0, ROWS_SUPER)], semO)
                cA = pltpu.async_copy(stA, outA.at[pl.ds(row0, ROWS_SUPER)], semO)
                cD = pltpu.async_copy(stD, outD.at[pl.ds(row0, ROWS_SUPER)], semO)
                cS.wait()
                cA.wait()
                cD.wait()

            return 0

        lax.fori_loop(0, NSUB, sub_body, 0)

    return sc_kernel(feat, nodes3, adj3, dis3)


def _tc_dense(selfr, adjsum, dissum, W_self, W_adj, W_dis, WC_w, b1, b2):
    BLK = 2000
    grid = (B // BLK,)

    def body(xs_ref, xa_ref, xd_ref, ws_ref, wa_ref, wd_ref, wc_ref,
             b1_ref, b2_ref, o_ref):
        dn = (((1,), (1,)), ((), ()))
        hs = lax.dot_general(xs_ref[...], ws_ref[...], dn,
                             preferred_element_type=jnp.float32)
        ha = lax.dot_general(xa_ref[...], wa_ref[...], dn,
                             preferred_element_type=jnp.float32) * (1.0 / NUM_WALKS)
        hd = lax.dot_general(xd_ref[...], wd_ref[...], dn,
                             preferred_element_type=jnp.float32) * (1.0 / NUM_WALKS)
        h = jnp.concatenate([hs, ha, hd], axis=-1) + b1_ref[...]
        g = lax.dot_general(h, wc_ref[...], dn,
                            preferred_element_type=jnp.float32) + b2_ref[...]
        g = jnp.where(g >= 0, g, 0.2 * g)
        nrm = jnp.sqrt(jnp.sum(g * g, axis=-1, keepdims=True))
        o_ref[...] = g / jnp.maximum(nrm, 1e-12)

    rep = lambda i: (0, 0)
    return pl.pallas_call(
        body,
        grid=grid,
        in_specs=[
            pl.BlockSpec((BLK, D_IN), lambda i: (i, 0)),
            pl.BlockSpec((BLK, D_IN), lambda i: (i, 0)),
            pl.BlockSpec((BLK, D_IN), lambda i: (i, 0)),
            pl.BlockSpec((D_OUT // 3, D_IN), rep),
            pl.BlockSpec((D_OUT // 3, D_IN), rep),
            pl.BlockSpec((D_OUT // 3, D_IN), rep),
            pl.BlockSpec((D_OUT, D_OUT), rep),
            pl.BlockSpec((1, D_OUT), rep),
            pl.BlockSpec((1, D_OUT), rep),
        ],
        out_specs=pl.BlockSpec((BLK, D_OUT), lambda i: (i, 0)),
        out_shape=jax.ShapeDtypeStruct((B, D_OUT), jnp.float32),
    )(selfr, adjsum, dissum, W_self, W_adj, W_dis, WC_w, b1, b2)


def kernel(feat, nodes, adj_neighbors, dis_neighbors,
           W_self, W_adj, W_dis, WC_w, WC_b, bias):
    nodes = nodes.astype(jnp.int32)
    adj_neighbors = adj_neighbors.astype(jnp.int32)
    dis_neighbors = dis_neighbors.astype(jnp.int32)

    pad = BP - B
    nodes_p = jnp.concatenate([nodes, jnp.zeros((pad,), jnp.int32)])
    adj_p = jnp.concatenate(
        [adj_neighbors.reshape(-1), jnp.zeros((pad * NUM_WALKS,), jnp.int32)])
    dis_p = jnp.concatenate(
        [dis_neighbors.reshape(-1), jnp.zeros((pad * NUM_WALKS,), jnp.int32)])

    nodes3 = nodes_p.reshape(NW, NSUPER, ROWS_SUPER)
    adj3 = adj_p.reshape(NW, NSUB, SUB * NUM_WALKS)
    dis3 = dis_p.reshape(NW, NSUB, SUB * NUM_WALKS)

    selfr, adjsum, dissum = _sc_aggregate(feat, nodes3, adj3, dis3)

    return _tc_dense(selfr, adjsum, dissum, W_self, W_adj, W_dis, WC_w,
                     bias[None, :], WC_b[None, :])


# speedup vs baseline: 2.9400x; 1.1700x over previous
import functools

import jax
import jax.numpy as jnp
from jax import lax
from jax.experimental import pallas as pl
from jax.experimental.pallas import tpu as pltpu
from jax.experimental.pallas import tpu_sc as plsc

N_NODES = 100000
D_IN = 128
D_OUT = 384
B = 16000
NUM_WALKS = 16

NC, NS, L = 2, 16, 16
NW = NC * NS
BP = 16384
RPW = BP // NW
SUB = 8
SUPER = 4
NSUB = RPW // SUB
ROWS_SUPER = SUB * SUPER
NSUPER = RPW // ROWS_SUPER


def _sc_aggregate(feat, nodes3, adj3, dis3):
    mesh = plsc.VectorSubcoreMesh(core_axis_name="c", subcore_axis_name="s")
    NB = SUB * 16

    @functools.partial(
        pl.kernel,
        mesh=mesh,
        out_type=[jax.ShapeDtypeStruct((BP, D_IN), jnp.float32)] * 3,
        scratch_types=[
            pltpu.VMEM((NSUPER, ROWS_SUPER), jnp.int32),
            pltpu.VMEM((NSUB, NB), jnp.int32),
            pltpu.VMEM((NSUB, NB), jnp.int32),
            pltpu.VMEM((NB, D_IN), jnp.float32),
            pltpu.VMEM((NB, D_IN), jnp.float32),
            pltpu.VMEM((NB, D_IN), jnp.float32),
            pltpu.VMEM((NB, D_IN), jnp.float32),
            pltpu.VMEM((ROWS_SUPER, D_IN), jnp.float32),
            pltpu.VMEM((ROWS_SUPER, D_IN), jnp.float32),
            pltpu.VMEM((ROWS_SUPER, D_IN), jnp.float32),
            pltpu.VMEM((ROWS_SUPER, D_IN), jnp.float32),
            pltpu.VMEM((ROWS_SUPER, D_IN), jnp.float32),
            pltpu.VMEM((ROWS_SUPER, D_IN), jnp.float32),
        ] + [pltpu.SemaphoreType.DMA] * 8,
    )
    def sc_kernel(feat_hbm, nodes_hbm, adj_hbm, dis_hbm,
                  outS, outA, outD,
                  idxS, idxA, idxD, bufA0, bufA1, bufD0, bufD1,
                  stS0, stS1, stA0, stA1, stD0, stD1,
                  semA0, semA1, semD0, semD1, semS0, semS1, semO0, semO1):
        wid = lax.axis_index("s") * NC + lax.axis_index("c")
        bufA = (bufA0, bufA1)
        bufD = (bufD0, bufD1)
        stS = (stS0, stS1)
        stA = (stA0, stA1)
        stD = (stD0, stD1)
        semA = (semA0, semA1)
        semD = (semD0, semD1)
        semS = (semS0, semS1)
        semO = (semO0, semO1)

        pltpu.sync_copy(nodes_hbm.at[wid], idxS)
        pltpu.sync_copy(adj_hbm.at[wid], idxA)
        pltpu.sync_copy(dis_hbm.at[wid], idxD)

        pltpu.async_copy(feat_hbm.at[idxA.at[0]], bufA0, semA0)
        pltpu.async_copy(feat_hbm.at[idxD.at[0]], bufD0, semD0)

        def reduce_sub(bA, bD, sA, sD, q):
            def row_body(r, _):
                srow = q * SUB + r
                base = r * 16
                for cc in range(D_IN // L):
                    col = pl.ds(cc * L, L)
                    accA = bA[base, col]
                    accD = bD[base, col]
                    for j in range(1, 16):
                        accA = accA + bA[base + j, col]
                        accD = accD + bD[base + j, col]
                    sA[srow, col] = accA
                    sD[srow, col] = accD
                return 0
            lax.fori_loop(0, SUB, row_body, 0)

        def drain_flush(h, s):
            row0 = wid * RPW + s * ROWS_SUPER
            pltpu.make_async_copy(stS[h], outS.at[pl.ds(row0, ROWS_SUPER)], semO[h]).wait()
            pltpu.make_async_copy(stA[h], outA.at[pl.ds(row0, ROWS_SUPER)], semO[h]).wait()
            pltpu.make_async_copy(stD[h], outD.at[pl.ds(row0, ROWS_SUPER)], semO[h]).wait()

        def pair_body(i, _):
            for h in range(2):
                s = i * 2 + h

                @pl.when(s >= 2)
                def _reclaim():
                    drain_flush(h, s - 2)

                pltpu.async_copy(feat_hbm.at[idxS.at[s]], stS[h], semS[h])

                for q in range(SUPER):
                    k = s * SUPER + q
                    p = q % 2

                    @pl.when(k + 1 < NSUB)
                    def _issue_next():
                        pltpu.async_copy(feat_hbm.at[idxA.at[k + 1]],
                                         bufA[1 - p], semA[1 - p])
                        pltpu.async_copy(feat_hbm.at[idxD.at[k + 1]],
                                         bufD[1 - p], semD[1 - p])

                    pltpu.make_async_copy(feat_hbm.at[idxA.at[k]],
                                          bufA[p], semA[p]).wait()
                    pltpu.make_async_copy(feat_hbm.at[idxD.at[k]],
                                          bufD[p], semD[p]).wait()
                    reduce_sub(bufA[p], bufD[p], stA[h], stD[h], q)

                row0 = wid * RPW + s * ROWS_SUPER
                pltpu.make_async_copy(feat_hbm.at[idxS.at[s]], stS[h], semS[h]).wait()
                pltpu.async_copy(stS[h], outS.at[pl.ds(row0, ROWS_SUPER)], semO[h])
                pltpu.async_copy(stA[h], outA.at[pl.ds(row0, ROWS_SUPER)], semO[h])
                pltpu.async_copy(stD[h], outD.at[pl.ds(row0, ROWS_SUPER)], semO[h])
            return 0

        lax.fori_loop(0, NSUPER // 2, pair_body, 0)

        drain_flush(0, NSUPER - 2)
        drain_flush(1, NSUPER - 1)

    return sc_kernel(feat, nodes3, adj3, dis3)


def _tc_dense(selfr, adjsum, dissum, W_self, W_adj, W_dis, WC_w, b1, b2):
    BLK = 2000
    grid = (B // BLK,)

    def body(xs_ref, xa_ref, xd_ref, ws_ref, wa_ref, wd_ref, wc_ref,
             b1_ref, b2_ref, o_ref):
        dn = (((1,), (1,)), ((), ()))
        hs = lax.dot_general(xs_ref[...], ws_ref[...], dn,
                             preferred_element_type=jnp.float32)
        ha = lax.dot_general(xa_ref[...], wa_ref[...], dn,
                             preferred_element_type=jnp.float32) * (1.0 / NUM_WALKS)
        hd = lax.dot_general(xd_ref[...], wd_ref[...], dn,
                             preferred_element_type=jnp.float32) * (1.0 / NUM_WALKS)
        h = jnp.concatenate([hs, ha, hd], axis=-1) + b1_ref[...]
        g = lax.dot_general(h, wc_ref[...], dn,
                            preferred_element_type=jnp.float32) + b2_ref[...]
        g = jnp.where(g >= 0, g, 0.2 * g)
        nrm = jnp.sqrt(jnp.sum(g * g, axis=-1, keepdims=True))
        o_ref[...] = g / jnp.maximum(nrm, 1e-12)

    rep = lambda i: (0, 0)
    return pl.pallas_call(
        body,
        grid=grid,
        in_specs=[
            pl.BlockSpec((BLK, D_IN), lambda i: (i, 0)),
            pl.BlockSpec((BLK, D_IN), lambda i: (i, 0)),
            pl.BlockSpec((BLK, D_IN), lambda i: (i, 0)),
            pl.BlockSpec((D_OUT // 3, D_IN), rep),
            pl.BlockSpec((D_OUT // 3, D_IN), rep),
            pl.BlockSpec((D_OUT // 3, D_IN), rep),
            pl.BlockSpec((D_OUT, D_OUT), rep),
            pl.BlockSpec((1, D_OUT), rep),
            pl.BlockSpec((1, D_OUT), rep),
        ],
        out_specs=pl.BlockSpec((BLK, D_OUT), lambda i: (i, 0)),
        out_shape=jax.ShapeDtypeStruct((B, D_OUT), jnp.float32),
    )(selfr, adjsum, dissum, W_self, W_adj, W_dis, WC_w, b1, b2)


def kernel(feat, nodes, adj_neighbors, dis_neighbors,
           W_self, W_adj, W_dis, WC_w, WC_b, bias):
    nodes = nodes.astype(jnp.int32)
    adj_neighbors = adj_neighbors.astype(jnp.int32)
    dis_neighbors = dis_neighbors.astype(jnp.int32)

    pad = BP - B
    nodes_p = jnp.concatenate([nodes, jnp.zeros((pad,), jnp.int32)])
    adj_p = jnp.concatenate(
        [adj_neighbors.reshape(-1), jnp.zeros((pad * NUM_WALKS,), jnp.int32)])
    dis_p = jnp.concatenate(
        [dis_neighbors.reshape(-1), jnp.zeros((pad * NUM_WALKS,), jnp.int32)])

    nodes3 = nodes_p.reshape(NW, NSUPER, ROWS_SUPER)
    adj3 = adj_p.reshape(NW, NSUB, SUB * NUM_WALKS)
    dis3 = dis_p.reshape(NW, NSUB, SUB * NUM_WALKS)

    selfr, adjsum, dissum = _sc_aggregate(feat, nodes3, adj3, dis3)

    return _tc_dense(selfr, adjsum, dissum, W_self, W_adj, W_dis, WC_w,
                     bias[None, :], WC_b[None, :])
